# baseline (speedup 1.0000x reference)
import jax
import jax.numpy as jnp
from jax import lax
from jax.experimental import pallas as pl
from jax.experimental.pallas import tpu as pltpu
from jax.experimental.pallas import tpu_sc as plsc

N = 10000
E = 320000
EMB = 128
HID = 64
D = 80
CHUNK = 128
NC = 2
NS = 16
NW = NC * NS
KPT = 80
NCHUNKS = NW * KPT
EPAD = NCHUNKS * CHUNK
NSLOT = 4
N_ACC = 10240
ROWS_PT = N_ACC // NS
ZROWS = 128

_f32 = jnp.float32


def _tables_body(h_ref, x_ref, w1a_ref, w1b_ref, b1_ref, ts_ref, td_ref):
    h = h_ref[...]
    nb = h.shape[0]
    pad = jnp.zeros((nb, D - HID - 3), _f32)
    a = jnp.dot(h, w1a_ref[...], preferred_element_type=_f32) + b1_ref[...]
    ts_ref[...] = jnp.concatenate([a, x_ref[...], pad], axis=1)
    b = jnp.dot(h, w1b_ref[...], preferred_element_type=_f32)
    td_ref[...] = jnp.concatenate([b, -x_ref[...], pad], axis=1)


def _build_tables(h, x, w1a, w1b, b1):
    nb = 2000
    grid = N // nb
    return pl.pallas_call(
        _tables_body,
        grid=(grid,),
        in_specs=[
            pl.BlockSpec((nb, EMB), lambda i: (i, 0)),
            pl.BlockSpec((nb, 3), lambda i: (i, 0)),
            pl.BlockSpec((EMB, HID), lambda i: (0, 0)),
            pl.BlockSpec((EMB, HID), lambda i: (0, 0)),
            pl.BlockSpec((1, HID), lambda i: (0, 0)),
        ],
        out_specs=[
            pl.BlockSpec((nb, D), lambda i: (i, 0)),
            pl.BlockSpec((nb, D), lambda i: (i, 0)),
        ],
        out_shape=[
            jax.ShapeDtypeStruct((N, D), _f32),
            jax.ShapeDtypeStruct((N, D), _f32),
        ],
    )(h, x, w1a, w1b, b1)


def _gather_body(tsrc, tdst, row2d, col2d, z_out, idx_r, idx_c, zbufs,
                 g1, g2, w):
    cid = lax.axis_index("c")
    sid = lax.axis_index("s")
    wid = sid * NC + cid
    chunk0 = wid * KPT

    pltpu.sync_copy(row2d.at[pl.ds(chunk0, KPT)], idx_r)
    pltpu.sync_copy(col2d.at[pl.ds(chunk0, KPT)], idx_c)

    def g1_start(k, s):
        pltpu.async_copy(tsrc.at[idx_r.at[k]], zbufs.at[s], g1.at[s])

    def g1_wait(k, s):
        pltpu.make_async_copy(tsrc.at[idx_r.at[k]], zbufs.at[s],
                              g1.at[s]).wait()

    def g2_start(k, s):
        pltpu.async_copy(tdst.at[idx_c.at[k]], zbufs.at[s], g2.at[s],
                         add=True)

    def g2_wait(k, s):
        pltpu.make_async_copy(tdst.at[idx_c.at[k]], zbufs.at[s],
                              g2.at[s]).wait()

    def w_start(k, s):
        base = (chunk0 + k) * CHUNK
        pltpu.async_copy(zbufs.at[s], z_out.at[pl.ds(base, CHUNK)], w.at[s])

    def w_wait(k, s):
        base = (chunk0 + k) * CHUNK
        pltpu.make_async_copy(zbufs.at[s], z_out.at[pl.ds(base, CHUNK)],
                              w.at[s]).wait()

    g1_start(0, 0)

    def body(kb, carry):
        for j in range(NSLOT):
            k = kb * NSLOT + j
            s = j
            snext = (j + 1) % NSLOT
            g1_wait(k, s)
            g2_start(k, s)

            @pl.when(k + 1 < KPT)
            def _():
                @pl.when(k >= NSLOT - 1)
                def _():
                    w_wait(k - (NSLOT - 1), snext)

                g1_start(k + 1, snext)

            g2_wait(k, s)
            w_start(k, s)
        return carry

    lax.fori_loop(0, KPT // NSLOT, body, 0)
    for s in range(NSLOT):
        w_wait(KPT - NSLOT + s, s)


def _gather_edges(tsrc, tdst, row2d, col2d):
    mesh = plsc.VectorSubcoreMesh(
        core_axis_name="c", subcore_axis_name="s", num_cores=NC, num_subcores=NS
    )
    return pl.kernel(
        _gather_body,
        out_type=jax.ShapeDtypeStruct((EPAD, D), _f32),
        mesh=mesh,
        compiler_params=pltpu.CompilerParams(use_tc_tiling_on_sc=False),
        scratch_types=[
            pltpu.VMEM((KPT, CHUNK), jnp.int32),
            pltpu.VMEM((KPT, CHUNK), jnp.int32),
            pltpu.VMEM((NSLOT, CHUNK, D), _f32),
            pltpu.SemaphoreType.DMA((NSLOT,)),
            pltpu.SemaphoreType.DMA((NSLOT,)),
            pltpu.SemaphoreType.DMA((NSLOT,)),
        ],
    )(tsrc, tdst, row2d, col2d)


def _edge_mlp_body(z_ref, s1_ref, sq_ref, s2_ref, srad_ref, w2_ref, b2_ref,
                   cw1_ref, cb1_ref, cw2_ref, cb2_ref, u_ref):
    z = z_ref[...]
    nb = z.shape[0]
    zz = z * z
    rad = jnp.dot(zz, srad_ref[...], preferred_element_type=_f32)
    z1 = jax.nn.relu(
        jnp.dot(z, s1_ref[...], preferred_element_type=_f32)
        + jnp.dot(zz, sq_ref[...], preferred_element_type=_f32)
    )
    msgs = jax.nn.relu(
        jnp.dot(z1, w2_ref[...], preferred_element_type=_f32) + b2_ref[...]
    )
    t = jax.nn.relu(
        jnp.dot(msgs, cw1_ref[...], preferred_element_type=_f32) + cb1_ref[...]
    )
    cc = jnp.dot(t, cw2_ref[...], preferred_element_type=_f32) + cb2_ref[...]
    scale = cc * lax.rsqrt(rad + 1e-8)
    cu = jnp.dot(z, s2_ref[...], preferred_element_type=_f32) * scale
    pad = jnp.zeros((nb, D - HID - 4), _f32)
    u_ref[...] = jnp.concatenate([msgs, cu, pad], axis=1)


def _edge_mlp(z, s1, sq, s2, srad, w2, b2, cw1, cb1, cw2, cb2):
    eb = 2048
    grid = EPAD // eb
    full = lambda r, c: pl.BlockSpec((r, c), lambda i: (0, 0))
    return pl.pallas_call(
        _edge_mlp_body,
        grid=(grid,),
        in_specs=[
            pl.BlockSpec((eb, D), lambda i: (i, 0)),
            full(D, HID), full(D, HID), full(D, 4), full(D, 1),
            full(HID, HID), full(1, HID),
            full(HID, HID), full(1, HID), full(HID, 1), full(1, 1),
        ],
        out_specs=pl.BlockSpec((eb, D), lambda i: (i, 0)),
        out_shape=jax.ShapeDtypeStruct((EPAD, D), _f32),
    )(z, s1, sq, s2, srad, w2, b2, cw1, cb1, cw2, cb2)


def _scatter_body(u, row2d, zeros_hbm, p_out, ubufs, idxb, zbuf, acc, r, ss):
    cid = lax.axis_index("c")
    sid = lax.axis_index("s")
    wid = sid * NC + cid
    chunk0 = wid * KPT
    row0 = sid * ROWS_PT

    pltpu.sync_copy(zeros_hbm, zbuf)
    for j in range(ROWS_PT // ZROWS):
        pltpu.sync_copy(zbuf, acc.at[pl.ds(row0 + j * ZROWS, ZROWS)])
    pltpu.sync_copy(row2d.at[pl.ds(chunk0, KPT)], idxb)
    plsc.subcore_barrier()

    def r_start(k, s):
        base = (chunk0 + k) * CHUNK
        pltpu.async_copy(u.at[pl.ds(base, CHUNK)], ubufs.at[s], r.at[s])

    def r_wait(k, s):
        base = (chunk0 + k) * CHUNK
        pltpu.make_async_copy(u.at[pl.ds(base, CHUNK)], ubufs.at[s],
                              r.at[s]).wait()

    def s_start(k, s):
        pltpu.async_copy(ubufs.at[s], acc.at[idxb.at[k]], ss.at[s], add=True)

    def s_wait(k, s):
        pltpu.make_async_copy(ubufs.at[s], acc.at[idxb.at[k]], ss.at[s]).wait()

    r_start(0, 0)

    def body(kb, carry):
        for j in range(NSLOT):
            k = kb * NSLOT + j
            s = j
            snext = (j + 1) % NSLOT
            r_wait(k, s)
            s_start(k, s)

            @pl.when(k + 1 < KPT)
            def _():
                @pl.when(k >= NSLOT - 1)
                def _():
                    s_wait(k - (NSLOT - 1), snext)

                r_start(k + 1, snext)
        return carry

    lax.fori_loop(0, KPT // NSLOT, body, 0)
    for s in range(NSLOT):
        s_wait(KPT - NSLOT + s, s)
    plsc.subcore_barrier()

    for j in range(ROWS_PT // ZROWS):
        rr = row0 + j * ZROWS
        pltpu.sync_copy(acc.at[pl.ds(rr, ZROWS)], zbuf)
        pltpu.sync_copy(zbuf, p_out.at[cid].at[pl.ds(rr, ZROWS)])


def _scatter_edges(u, row2d, zeros_hbm):
    mesh = plsc.VectorSubcoreMesh(
        core_axis_name="c", subcore_axis_name="s", num_cores=NC, num_subcores=NS
    )
    return pl.kernel(
        _scatter_body,
        out_type=jax.ShapeDtypeStruct((NC, N_ACC, D), _f32),
        mesh=mesh,
        compiler_params=pltpu.CompilerParams(use_tc_tiling_on_sc=False),
        scratch_types=[
            pltpu.VMEM((NSLOT, CHUNK, D), _f32),
            pltpu.VMEM((KPT, CHUNK), jnp.int32),
            pltpu.VMEM((ZROWS, D), _f32),
            pltpu.VMEM_SHARED((N_ACC, D), _f32),
            pltpu.SemaphoreType.DMA((NSLOT,)),
            pltpu.SemaphoreType.DMA((NSLOT,)),
        ],
    )(u, row2d, zeros_hbm)


def _node_mlp_body(p_ref, h_ref, x_ref, w1f_ref, w1h_ref, b1_ref, w2_ref,
                   b2_ref, sx_ref, xo_ref, ho_ref):
    g = p_ref[0] + p_ref[1]
    h = h_ref[...]
    t = jax.nn.relu(
        jnp.dot(g, w1f_ref[...], preferred_element_type=_f32)
        + jnp.dot(h, w1h_ref[...], preferred_element_type=_f32)
        + b1_ref[...]
    )
    ho_ref[...] = h + jnp.dot(t, w2_ref[...], preferred_element_type=_f32) \
        + b2_ref[...]
    xo_ref[...] = x_ref[...] + jnp.dot(g, sx_ref[...],
                                       preferred_element_type=_f32)


def _node_mlp(p, h, x, w1f_ext, w1h, b1, w2, b2, sx):
    nb = 2000
    grid = N // nb
    full = lambda r, c: pl.BlockSpec((r, c), lambda i: (0, 0))
    return pl.pallas_call(
        _node_mlp_body,
        grid=(grid,),
        in_specs=[
            pl.BlockSpec((NC, nb, D), lambda i: (0, i, 0)),
            pl.BlockSpec((nb, EMB), lambda i: (i, 0)),
            pl.BlockSpec((nb, 3), lambda i: (i, 0)),
            full(D, HID), full(EMB, HID), full(1, HID),
            full(HID, EMB), full(1, EMB), full(D, 3),
        ],
        out_specs=[
            pl.BlockSpec((nb, 3), lambda i: (i, 0)),
            pl.BlockSpec((nb, EMB), lambda i: (i, 0)),
        ],
        out_shape=[
            jax.ShapeDtypeStruct((N, 3), _f32),
            jax.ShapeDtypeStruct((N, EMB), _f32),
        ],
    )(p, h, x, w1f_ext, w1h, b1, w2, b2, sx)


def kernel(x, h, edge_index, msg_W1, msg_b1, msg_W2, msg_b2, coord_W1,
           coord_b1, coord_W2, coord_b2, node_W1, node_b1, node_W2, node_b2):
    ei = edge_index.astype(jnp.int32)
    row = ei[0]
    col = ei[1]
    npad = EPAD - E
    zpad = jnp.zeros((npad,), jnp.int32)
    row_g2d = jnp.concatenate([row, zpad]).reshape(NCHUNKS, CHUNK)
    col_g2d = jnp.concatenate([col, zpad]).reshape(NCHUNKS, CHUNK)
    dummy = N + (jnp.arange(npad, dtype=jnp.int32) % (N_ACC - N))
    row_s2d = jnp.concatenate([row, dummy]).reshape(NCHUNKS, CHUNK)

    w1a = msg_W1[:EMB]
    w1b = msg_W1[EMB:2 * EMB]
    w1c = msg_W1[2 * EMB]

    tsrc, tdst = _build_tables(h, x, w1a, w1b, msg_b1.reshape(1, HID))
    z = _gather_edges(tsrc, tdst, row_g2d, col_g2d)

    eye = jnp.eye(D, dtype=_f32)
    s1 = eye[:, :HID]
    sq = jnp.zeros((D, HID), _f32).at[HID:HID + 3].set(
        jnp.broadcast_to(w1c, (3, HID)))
    s2 = eye[:, HID:HID + 4]
    srad = jnp.zeros((D, 1), _f32).at[HID:HID + 3].set(1.0)

    u = _edge_mlp(z, s1, sq, s2, srad, msg_W2, msg_b2.reshape(1, HID),
                  coord_W1, coord_b1.reshape(1, HID), coord_W2,
                  coord_b2.reshape(1, 1))

    p = _scatter_edges(u, row_s2d, jnp.zeros((ZROWS, D), _f32))

    w1f_ext = jnp.zeros((D, HID), _f32).at[:HID].set(node_W1[:HID])
    sx = jnp.zeros((D, 3), _f32).at[HID:HID + 3].set(jnp.eye(3, dtype=_f32))

    x_new, h_new = _node_mlp(p, h, x, w1f_ext, node_W1[HID:],
                             node_b1.reshape(1, HID), node_W2,
                             node_b2.reshape(1, EMB), sx)
    return (x_new, h_new)


# speedup vs baseline: 1.0086x; 1.0086x over previous
import jax
import jax.numpy as jnp
from jax import lax
from jax.experimental import pallas as pl
from jax.experimental.pallas import tpu as pltpu
from jax.experimental.pallas import tpu_sc as plsc

N = 10000
E = 320000
EMB = 128
HID = 64
D = 80
CHUNK = 128
NC = 2
NS = 16
NW = NC * NS
KPT = 80
NCHUNKS = NW * KPT
EPAD = NCHUNKS * CHUNK
NSLOT = 4
N_ACC = 10240
ROWS_PT = N_ACC // NS
ZROWS = 128

_f32 = jnp.float32


def _tables_body(h_ref, x_ref, w_ref, b_ref, sgn_ref, t_ref):
    h = h_ref[...]
    nb = h.shape[0]
    pad = jnp.zeros((nb, D - HID - 3), _f32)
    a = jnp.dot(h, w_ref[0], preferred_element_type=_f32) + b_ref[0]
    t_ref[...] = jnp.concatenate([a, sgn_ref[0, 0, 0] * x_ref[...], pad],
                                 axis=1)


def _build_tables(h, x, w1ab, b1z, sgn):
    nb = 2000
    grid = N // nb
    return pl.pallas_call(
        _tables_body,
        grid=(2, grid),
        in_specs=[
            pl.BlockSpec((nb, EMB), lambda t, i: (i, 0)),
            pl.BlockSpec((nb, 3), lambda t, i: (i, 0)),
            pl.BlockSpec((1, EMB, HID), lambda t, i: (t, 0, 0)),
            pl.BlockSpec((1, 1, HID), lambda t, i: (t, 0, 0)),
            pl.BlockSpec((1, 1, 1), lambda t, i: (t, 0, 0)),
        ],
        out_specs=pl.BlockSpec((nb, D), lambda t, i: (t * grid + i, 0)),
        out_shape=jax.ShapeDtypeStruct((2 * N, D), _f32),
    )(h, x, w1ab, b1z, sgn)


def _gather_body(tcomb, row2d, col2d, z_out, idx_r, idx_c, zbufs,
                 tab_s, g1, g2, w):
    cid = lax.axis_index("c")
    sid = lax.axis_index("s")
    wid = sid * NC + cid
    chunk0 = wid * KPT

    rows_st = N // NS
    r0 = sid * rows_st
    pltpu.sync_copy(tcomb.at[pl.ds(N + r0, rows_st)],
                    tab_s.at[pl.ds(r0, rows_st)])

    pltpu.sync_copy(row2d.at[pl.ds(chunk0, KPT)], idx_r)
    pltpu.sync_copy(col2d.at[pl.ds(chunk0, KPT)], idx_c)
    plsc.subcore_barrier()

    def g1_start(k, s):
        pltpu.async_copy(tcomb.at[idx_r.at[k]], zbufs.at[s], g1.at[s])

    def g1_wait(k, s):
        pltpu.make_async_copy(tcomb.at[idx_r.at[k]], zbufs.at[s],
                              g1.at[s]).wait()

    def g2_start(k, s):
        pltpu.async_copy(tab_s.at[idx_c.at[k]], zbufs.at[s], g2.at[s],
                         add=True)

    def g2_wait(k, s):
        pltpu.make_async_copy(tab_s.at[idx_c.at[k]], zbufs.at[s],
                              g2.at[s]).wait()

    def w_start(k, s):
        base = (chunk0 + k) * CHUNK
        pltpu.async_copy(zbufs.at[s], z_out.at[pl.ds(base, CHUNK)], w.at[s])

    def w_wait(k, s):
        base = (chunk0 + k) * CHUNK
        pltpu.make_async_copy(zbufs.at[s], z_out.at[pl.ds(base, CHUNK)],
                              w.at[s]).wait()

    g1_start(0, 0)

    def body(kb, carry):
        for j in range(NSLOT):
            k = kb * NSLOT + j
            s = j
            snext = (j + 1) % NSLOT
            g1_wait(k, s)
            g2_start(k, s)

            @pl.when(k + 1 < KPT)
            def _():
                @pl.when(k >= NSLOT - 1)
                def _():
                    w_wait(k - (NSLOT - 1), snext)

                g1_start(k + 1, snext)

            g2_wait(k, s)
            w_start(k, s)
        return carry

    lax.fori_loop(0, KPT // NSLOT, body, 0)
    for s in range(NSLOT):
        w_wait(KPT - NSLOT + s, s)


def _gather_edges(tcomb, row2d, col2d):
    mesh = plsc.VectorSubcoreMesh(
        core_axis_name="c", subcore_axis_name="s", num_cores=NC, num_subcores=NS
    )
    return pl.kernel(
        _gather_body,
        out_type=jax.ShapeDtypeStruct((EPAD, D), _f32),
        mesh=mesh,
        compiler_params=pltpu.CompilerParams(use_tc_tiling_on_sc=False),
        scratch_types=[
            pltpu.VMEM((KPT, CHUNK), jnp.int32),
            pltpu.VMEM((KPT, CHUNK), jnp.int32),
            pltpu.VMEM((NSLOT, CHUNK, D), _f32),
            pltpu.VMEM_SHARED((N, D), _f32),
            pltpu.SemaphoreType.DMA((NSLOT,)),
            pltpu.SemaphoreType.DMA((NSLOT,)),
            pltpu.SemaphoreType.DMA((NSLOT,)),
        ],
    )(tcomb, row2d, col2d)


def _edge_mlp_body(z_ref, s1_ref, sq_ref, s2_ref, srad_ref, w2_ref, b2_ref,
                   cw1_ref, cb1_ref, cw2_ref, cb2_ref, u_ref):
    z = z_ref[...]
    nb = z.shape[0]
    zz = z * z
    rad = jnp.dot(zz, srad_ref[...], preferred_element_type=_f32)
    z1 = jax.nn.relu(
        jnp.dot(z, s1_ref[...], preferred_element_type=_f32)
        + jnp.dot(zz, sq_ref[...], preferred_element_type=_f32)
    )
    msgs = jax.nn.relu(
        jnp.dot(z1, w2_ref[...], preferred_element_type=_f32) + b2_ref[...]
    )
    t = jax.nn.relu(
        jnp.dot(msgs, cw1_ref[...], preferred_element_type=_f32) + cb1_ref[...]
    )
    cc = jnp.dot(t, cw2_ref[...], preferred_element_type=_f32) + cb2_ref[...]
    scale = cc * lax.rsqrt(rad + 1e-8)
    cu = jnp.dot(z, s2_ref[...], preferred_element_type=_f32) * scale
    pad = jnp.zeros((nb, D - HID - 4), _f32)
    u_ref[...] = jnp.concatenate([msgs, cu, pad], axis=1)


def _edge_mlp(z, s1, sq, s2, srad, w2, b2, cw1, cb1, cw2, cb2):
    eb = 2048
    grid = EPAD // eb
    full = lambda r, c: pl.BlockSpec((r, c), lambda i: (0, 0))
    return pl.pallas_call(
        _edge_mlp_body,
        grid=(grid,),
        in_specs=[
            pl.BlockSpec((eb, D), lambda i: (i, 0)),
            full(D, HID), full(D, HID), full(D, 4), full(D, 1),
            full(HID, HID), full(1, HID),
            full(HID, HID), full(1, HID), full(HID, 1), full(1, 1),
        ],
        out_specs=pl.BlockSpec((eb, D), lambda i: (i, 0)),
        out_shape=jax.ShapeDtypeStruct((EPAD, D), _f32),
    )(z, s1, sq, s2, srad, w2, b2, cw1, cb1, cw2, cb2)


def _scatter_body(u, row2d, zeros_hbm, p_out, ubufs, idxb, zbuf, acc, r, ss):
    cid = lax.axis_index("c")
    sid = lax.axis_index("s")
    wid = sid * NC + cid
    chunk0 = wid * KPT
    row0 = sid * ROWS_PT

    pltpu.sync_copy(zeros_hbm, zbuf)
    for j in range(ROWS_PT // ZROWS):
        pltpu.sync_copy(zbuf, acc.at[pl.ds(row0 + j * ZROWS, ZROWS)])
    pltpu.sync_copy(row2d.at[pl.ds(chunk0, KPT)], idxb)
    plsc.subcore_barrier()

    def r_start(k, s):
        base = (chunk0 + k) * CHUNK
        pltpu.async_copy(u.at[pl.ds(base, CHUNK)], ubufs.at[s], r.at[s])

    def r_wait(k, s):
        base = (chunk0 + k) * CHUNK
        pltpu.make_async_copy(u.at[pl.ds(base, CHUNK)], ubufs.at[s],
                              r.at[s]).wait()

    def s_start(k, s):
        pltpu.async_copy(ubufs.at[s], acc.at[idxb.at[k]], ss.at[s], add=True)

    def s_wait(k, s):
        pltpu.make_async_copy(ubufs.at[s], acc.at[idxb.at[k]], ss.at[s]).wait()

    r_start(0, 0)

    def body(kb, carry):
        for j in range(NSLOT):
            k = kb * NSLOT + j
            s = j
            snext = (j + 1) % NSLOT
            r_wait(k, s)
            s_start(k, s)

            @pl.when(k + 1 < KPT)
            def _():
                @pl.when(k >= NSLOT - 1)
                def _():
                    s_wait(k - (NSLOT - 1), snext)

                r_start(k + 1, snext)
        return carry

    lax.fori_loop(0, KPT // NSLOT, body, 0)
    for s in range(NSLOT):
        s_wait(KPT - NSLOT + s, s)
    plsc.subcore_barrier()

    for j in range(ROWS_PT // ZROWS):
        rr = row0 + j * ZROWS
        pltpu.sync_copy(acc.at[pl.ds(rr, ZROWS)], zbuf)
        pltpu.sync_copy(zbuf, p_out.at[cid].at[pl.ds(rr, ZROWS)])


def _scatter_edges(u, row2d, zeros_hbm):
    mesh = plsc.VectorSubcoreMesh(
        core_axis_name="c", subcore_axis_name="s", num_cores=NC, num_subcores=NS
    )
    return pl.kernel(
        _scatter_body,
        out_type=jax.ShapeDtypeStruct((NC, N_ACC, D), _f32),
        mesh=mesh,
        compiler_params=pltpu.CompilerParams(use_tc_tiling_on_sc=False),
        scratch_types=[
            pltpu.VMEM((NSLOT, CHUNK, D), _f32),
            pltpu.VMEM((KPT, CHUNK), jnp.int32),
            pltpu.VMEM((ZROWS, D), _f32),
            pltpu.VMEM_SHARED((N_ACC, D), _f32),
            pltpu.SemaphoreType.DMA((NSLOT,)),
            pltpu.SemaphoreType.DMA((NSLOT,)),
        ],
    )(u, row2d, zeros_hbm)


def _node_mlp_body(p_ref, h_ref, x_ref, w1f_ref, w1h_ref, b1_ref, w2_ref,
                   b2_ref, sx_ref, xo_ref, ho_ref):
    g = p_ref[0] + p_ref[1]
    h = h_ref[...]
    t = jax.nn.relu(
        jnp.dot(g, w1f_ref[...], preferred_element_type=_f32)
        + jnp.dot(h, w1h_ref[...], preferred_element_type=_f32)
        + b1_ref[...]
    )
    ho_ref[...] = h + jnp.dot(t, w2_ref[...], preferred_element_type=_f32) \
        + b2_ref[...]
    xo_ref[...] = x_ref[...] + jnp.dot(g, sx_ref[...],
                                       preferred_element_type=_f32)


def _node_mlp(p, h, x, w1f_ext, w1h, b1, w2, b2, sx):
    nb = 2000
    grid = N // nb
    full = lambda r, c: pl.BlockSpec((r, c), lambda i: (0, 0))
    return pl.pallas_call(
        _node_mlp_body,
        grid=(grid,),
        in_specs=[
            pl.BlockSpec((NC, nb, D), lambda i: (0, i, 0)),
            pl.BlockSpec((nb, EMB), lambda i: (i, 0)),
            pl.BlockSpec((nb, 3), lambda i: (i, 0)),
            full(D, HID), full(EMB, HID), full(1, HID),
            full(HID, EMB), full(1, EMB), full(D, 3),
        ],
        out_specs=[
            pl.BlockSpec((nb, 3), lambda i: (i, 0)),
            pl.BlockSpec((nb, EMB), lambda i: (i, 0)),
        ],
        out_shape=[
            jax.ShapeDtypeStruct((N, 3), _f32),
            jax.ShapeDtypeStruct((N, EMB), _f32),
        ],
    )(p, h, x, w1f_ext, w1h, b1, w2, b2, sx)


def kernel(x, h, edge_index, msg_W1, msg_b1, msg_W2, msg_b2, coord_W1,
           coord_b1, coord_W2, coord_b2, node_W1, node_b1, node_W2, node_b2):
    ei = edge_index.astype(jnp.int32)
    row = ei[0]
    col = ei[1]
    npad = EPAD - E
    zpad = jnp.zeros((npad,), jnp.int32)
    row_g2d = jnp.concatenate([row, zpad]).reshape(NCHUNKS, CHUNK)
    col_g2d = jnp.concatenate([col, zpad]).reshape(NCHUNKS, CHUNK)
    dummy = N + (jnp.arange(npad, dtype=jnp.int32) % (N_ACC - N))
    row_s2d = jnp.concatenate([row, dummy]).reshape(NCHUNKS, CHUNK)

    w1ab = jnp.stack([msg_W1[:EMB], msg_W1[EMB:2 * EMB]])
    w1c = msg_W1[2 * EMB]
    b1z = jnp.stack([msg_b1.reshape(1, HID), jnp.zeros((1, HID), _f32)])
    sgn = jnp.array([1.0, -1.0], _f32).reshape(2, 1, 1)

    tcomb = _build_tables(h, x, w1ab, b1z, sgn)
    z = _gather_edges(tcomb, row_g2d, col_g2d)

    eye = jnp.eye(D, dtype=_f32)
    s1 = eye[:, :HID]
    sq = jnp.zeros((D, HID), _f32).at[HID:HID + 3].set(
        jnp.broadcast_to(w1c, (3, HID)))
    s2 = eye[:, HID:HID + 4]
    srad = jnp.zeros((D, 1), _f32).at[HID:HID + 3].set(1.0)

    u = _edge_mlp(z, s1, sq, s2, srad, msg_W2, msg_b2.reshape(1, HID),
                  coord_W1, coord_b1.reshape(1, HID), coord_W2,
                  coord_b2.reshape(1, 1))

    p = _scatter_edges(u, row_s2d, jnp.zeros((ZROWS, D), _f32))

    w1f_ext = jnp.zeros((D, HID), _f32).at[:HID].set(node_W1[:HID])
    sx = jnp.zeros((D, 3), _f32).at[HID:HID + 3].set(jnp.eye(3, dtype=_f32))

    x_new, h_new = _node_mlp(p, h, x, w1f_ext, node_W1[HID:],
                             node_b1.reshape(1, HID), node_W2,
                             node_b2.reshape(1, EMB), sx)
    return (x_new, h_new)


# speedup vs baseline: 1.4021x; 1.3900x over previous
import jax
import jax.numpy as jnp
from jax import lax
from jax.experimental import pallas as pl
from jax.experimental.pallas import tpu as pltpu
from jax.experimental.pallas import tpu_sc as plsc

N = 10000
E = 320000
EMB = 128
HID = 64
D = 80
DZ = 128
CHUNK = 128
CHUNK_G = 256
NC = 2
NS = 16
NW = NC * NS
KPT = 80
NCHUNKS = NW * KPT
EPAD = NCHUNKS * CHUNK
KPT_G = EPAD // (NW * CHUNK_G)
NSLOT = 4
N_ACC = 10240
ROWS_PT = N_ACC // NS
ZROWS = 128

_f32 = jnp.float32


def _tables_body(h_ref, x_ref, w_ref, b_ref, sgn_ref, t_ref):
    h = h_ref[...]
    nb = h.shape[0]
    pad = jnp.zeros((nb, D - HID - 3), _f32)
    a = jnp.dot(h, w_ref[0], preferred_element_type=_f32) + b_ref[0]
    t_ref[...] = jnp.concatenate([a, sgn_ref[0, 0, 0] * x_ref[...], pad],
                                 axis=1)


def _build_tables(h, x, w1ab, b1z, sgn):
    nb = 2000
    grid = N // nb
    return pl.pallas_call(
        _tables_body,
        grid=(2, grid),
        in_specs=[
            pl.BlockSpec((nb, EMB), lambda t, i: (i, 0)),
            pl.BlockSpec((nb, 3), lambda t, i: (i, 0)),
            pl.BlockSpec((1, EMB, HID), lambda t, i: (t, 0, 0)),
            pl.BlockSpec((1, 1, HID), lambda t, i: (t, 0, 0)),
            pl.BlockSpec((1, 1, 1), lambda t, i: (t, 0, 0)),
        ],
        out_specs=pl.BlockSpec((nb, D), lambda t, i: (t * grid + i, 0)),
        out_shape=jax.ShapeDtypeStruct((2 * N, D), _f32),
    )(h, x, w1ab, b1z, sgn)


def _gather_body(tcomb, row2d, col2d, z_out, idx_r, idx_c, zbufs,
                 g1, g2, w):
    cid = lax.axis_index("c")
    sid = lax.axis_index("s")
    wid = sid * NC + cid
    chunk0 = wid * KPT_G

    pltpu.sync_copy(row2d.at[pl.ds(chunk0, KPT_G)], idx_r)
    pltpu.sync_copy(col2d.at[pl.ds(chunk0, KPT_G)], idx_c)

    def g1_start(k, s):
        pltpu.async_copy(tcomb.at[idx_r.at[k]], zbufs.at[s], g1.at[s])

    def g1_wait(k, s):
        pltpu.make_async_copy(tcomb.at[idx_r.at[k]], zbufs.at[s],
                              g1.at[s]).wait()

    def g2_start(k, s):
        pltpu.async_copy(tcomb.at[idx_c.at[k]], zbufs.at[s], g2.at[s],
                         add=True)

    def g2_wait(k, s):
        pltpu.make_async_copy(tcomb.at[idx_c.at[k]], zbufs.at[s],
                              g2.at[s]).wait()

    def _wdst(k):
        base = (chunk0 + k) * CHUNK_G
        return z_out.at[pl.ds(base, CHUNK_G)].at[:, pl.ds(0, D)]

    def w_start(k, s):
        pltpu.async_copy(zbufs.at[s], _wdst(k), w.at[s])

    def w_wait(k, s):
        pltpu.make_async_copy(zbufs.at[s], _wdst(k), w.at[s]).wait()

    g1_start(0, 0)

    def body(kb, carry):
        for j in range(NSLOT):
            k = kb * NSLOT + j
            s = j
            snext = (j + 1) % NSLOT
            g1_wait(k, s)
            g2_start(k, s)

            @pl.when(k + 1 < KPT_G)
            def _():
                @pl.when(k >= NSLOT - 1)
                def _():
                    w_wait(k - (NSLOT - 1), snext)

                g1_start(k + 1, snext)

            g2_wait(k, s)
            w_start(k, s)
        return carry

    lax.fori_loop(0, KPT_G // NSLOT, body, 0)
    for s in range(NSLOT):
        w_wait(KPT_G - NSLOT + s, s)


def _gather_edges(tcomb, row2d, col2d):
    mesh = plsc.VectorSubcoreMesh(
        core_axis_name="c", subcore_axis_name="s", num_cores=NC, num_subcores=NS
    )
    return pl.kernel(
        _gather_body,
        out_type=jax.ShapeDtypeStruct((EPAD, DZ), _f32),
        mesh=mesh,
        compiler_params=pltpu.CompilerParams(use_tc_tiling_on_sc=False),
        scratch_types=[
            pltpu.VMEM((KPT_G, CHUNK_G), jnp.int32),
            pltpu.VMEM((KPT_G, CHUNK_G), jnp.int32),
            pltpu.VMEM((NSLOT, CHUNK_G, D), _f32),
            pltpu.SemaphoreType.DMA((NSLOT,)),
            pltpu.SemaphoreType.DMA((NSLOT,)),
            pltpu.SemaphoreType.DMA((NSLOT,)),
        ],
    )(tcomb, row2d, col2d)


def _edge_mlp_body(z_ref, s1_ref, sq_ref, s2_ref, srad_ref, w2_ref, b2_ref,
                   cw1_ref, cb1_ref, cw2_ref, cb2_ref, u_ref):
    zr = z_ref[...]
    nb = zr.shape[0]
    lane = lax.broadcasted_iota(jnp.int32, (nb, DZ), 1)
    z = jnp.where(lane < D, zr, 0.0)
    zz = z * z
    rad = jnp.dot(zz, srad_ref[...], preferred_element_type=_f32)
    z1 = jax.nn.relu(
        jnp.dot(z, s1_ref[...], preferred_element_type=_f32)
        + jnp.dot(zz, sq_ref[...], preferred_element_type=_f32)
    )
    msgs = jax.nn.relu(
        jnp.dot(z1, w2_ref[...], preferred_element_type=_f32) + b2_ref[...]
    )
    t = jax.nn.relu(
        jnp.dot(msgs, cw1_ref[...], preferred_element_type=_f32) + cb1_ref[...]
    )
    cc = jnp.dot(t, cw2_ref[...], preferred_element_type=_f32) + cb2_ref[...]
    scale = cc * lax.rsqrt(rad + 1e-8)
    cu = jnp.dot(z, s2_ref[...], preferred_element_type=_f32) * scale
    pad = jnp.zeros((nb, DZ - HID - 4), _f32)
    u_ref[...] = jnp.concatenate([msgs, cu, pad], axis=1)


def _edge_mlp(z, s1, sq, s2, srad, w2, b2, cw1, cb1, cw2, cb2):
    eb = 2048
    grid = EPAD // eb
    full = lambda r, c: pl.BlockSpec((r, c), lambda i: (0, 0))
    return pl.pallas_call(
        _edge_mlp_body,
        grid=(grid,),
        in_specs=[
            pl.BlockSpec((eb, DZ), lambda i: (i, 0)),
            full(DZ, HID), full(DZ, HID), full(DZ, 4), full(DZ, 1),
            full(HID, HID), full(1, HID),
            full(HID, HID), full(1, HID), full(HID, 1), full(1, 1),
        ],
        out_specs=pl.BlockSpec((eb, DZ), lambda i: (i, 0)),
        out_shape=jax.ShapeDtypeStruct((EPAD, DZ), _f32),
    )(z, s1, sq, s2, srad, w2, b2, cw1, cb1, cw2, cb2)


def _scatter_body(u, row2d, zeros_hbm, p_out, ubufs, idxb, zbuf, acc, r, ss):
    cid = lax.axis_index("c")
    sid = lax.axis_index("s")
    wid = sid * NC + cid
    chunk0 = wid * KPT
    row0 = sid * ROWS_PT

    pltpu.sync_copy(zeros_hbm, zbuf)
    for j in range(ROWS_PT // ZROWS):
        pltpu.sync_copy(zbuf, acc.at[pl.ds(row0 + j * ZROWS, ZROWS)])
    pltpu.sync_copy(row2d.at[pl.ds(chunk0, KPT)], idxb)
    plsc.subcore_barrier()

    def _rsrc(k):
        base = (chunk0 + k) * CHUNK
        return u.at[pl.ds(base, CHUNK)].at[:, pl.ds(0, D)]

    def r_start(k, s):
        pltpu.async_copy(_rsrc(k), ubufs.at[s], r.at[s])

    def r_wait(k, s):
        pltpu.make_async_copy(_rsrc(k), ubufs.at[s], r.at[s]).wait()

    def s_start(k, s):
        pltpu.async_copy(ubufs.at[s], acc.at[idxb.at[k]], ss.at[s], add=True)

    def s_wait(k, s):
        pltpu.make_async_copy(ubufs.at[s], acc.at[idxb.at[k]], ss.at[s]).wait()

    r_start(0, 0)

    def body(kb, carry):
        for j in range(NSLOT):
            k = kb * NSLOT + j
            s = j
            snext = (j + 1) % NSLOT
            r_wait(k, s)
            s_start(k, s)

            @pl.when(k + 1 < KPT)
            def _():
                @pl.when(k >= NSLOT - 1)
                def _():
                    s_wait(k - (NSLOT - 1), snext)

                r_start(k + 1, snext)
        return carry

    lax.fori_loop(0, KPT // NSLOT, body, 0)
    for s in range(NSLOT):
        s_wait(KPT - NSLOT + s, s)
    plsc.subcore_barrier()

    for j in range(ROWS_PT // ZROWS):
        rr = row0 + j * ZROWS
        pltpu.sync_copy(acc.at[pl.ds(rr, ZROWS)], zbuf)
        pltpu.sync_copy(zbuf,
                        p_out.at[cid].at[pl.ds(rr, ZROWS)].at[:, pl.ds(0, D)])


def _scatter_edges(u, row2d, zeros_hbm):
    mesh = plsc.VectorSubcoreMesh(
        core_axis_name="c", subcore_axis_name="s", num_cores=NC, num_subcores=NS
    )
    return pl.kernel(
        _scatter_body,
        out_type=jax.ShapeDtypeStruct((NC, N_ACC, DZ), _f32),
        mesh=mesh,
        compiler_params=pltpu.CompilerParams(use_tc_tiling_on_sc=False),
        scratch_types=[
            pltpu.VMEM((NSLOT, CHUNK, D), _f32),
            pltpu.VMEM((KPT, CHUNK), jnp.int32),
            pltpu.VMEM((ZROWS, D), _f32),
            pltpu.VMEM_SHARED((N_ACC, D), _f32),
            pltpu.SemaphoreType.DMA((NSLOT,)),
            pltpu.SemaphoreType.DMA((NSLOT,)),
        ],
    )(u, row2d, zeros_hbm)


def _node_mlp_body(p_ref, h_ref, x_ref, w1f_ref, w1h_ref, b1_ref, w2_ref,
                   b2_ref, sx_ref, xo_ref, ho_ref):
    gr = p_ref[0] + p_ref[1]
    lane = lax.broadcasted_iota(jnp.int32, gr.shape, 1)
    g = jnp.where(lane < D, gr, 0.0)
    h = h_ref[...]
    t = jax.nn.relu(
        jnp.dot(g, w1f_ref[...], preferred_element_type=_f32)
        + jnp.dot(h, w1h_ref[...], preferred_element_type=_f32)
        + b1_ref[...]
    )
    ho_ref[...] = h + jnp.dot(t, w2_ref[...], preferred_element_type=_f32) \
        + b2_ref[...]
    xo_ref[...] = x_ref[...] + jnp.dot(g, sx_ref[...],
                                       preferred_element_type=_f32)


def _node_mlp(p, h, x, w1f_ext, w1h, b1, w2, b2, sx):
    nb = 2000
    grid = N // nb
    full = lambda r, c: pl.BlockSpec((r, c), lambda i: (0, 0))
    return pl.pallas_call(
        _node_mlp_body,
        grid=(grid,),
        in_specs=[
            pl.BlockSpec((NC, nb, DZ), lambda i: (0, i, 0)),
            pl.BlockSpec((nb, EMB), lambda i: (i, 0)),
            pl.BlockSpec((nb, 3), lambda i: (i, 0)),
            full(DZ, HID), full(EMB, HID), full(1, HID),
            full(HID, EMB), full(1, EMB), full(DZ, 3),
        ],
        out_specs=[
            pl.BlockSpec((nb, 3), lambda i: (i, 0)),
            pl.BlockSpec((nb, EMB), lambda i: (i, 0)),
        ],
        out_shape=[
            jax.ShapeDtypeStruct((N, 3), _f32),
            jax.ShapeDtypeStruct((N, EMB), _f32),
        ],
    )(p, h, x, w1f_ext, w1h, b1, w2, b2, sx)


def kernel(x, h, edge_index, msg_W1, msg_b1, msg_W2, msg_b2, coord_W1,
           coord_b1, coord_W2, coord_b2, node_W1, node_b1, node_W2, node_b2):
    ei = edge_index.astype(jnp.int32)
    row = ei[0]
    col = ei[1]
    npad = EPAD - E
    zpad = jnp.zeros((npad,), jnp.int32)
    row_g2d = jnp.concatenate([row, zpad]).reshape(EPAD // CHUNK_G, CHUNK_G)
    col_g2d = (jnp.concatenate([col, zpad]) + N).reshape(EPAD // CHUNK_G,
                                                         CHUNK_G)
    dummy = N + (jnp.arange(npad, dtype=jnp.int32) % (N_ACC - N))
    row_s2d = jnp.concatenate([row, dummy]).reshape(NCHUNKS, CHUNK)

    w1ab = jnp.stack([msg_W1[:EMB], msg_W1[EMB:2 * EMB]])
    w1c = msg_W1[2 * EMB]
    b1z = jnp.stack([msg_b1.reshape(1, HID), jnp.zeros((1, HID), _f32)])
    sgn = jnp.array([1.0, -1.0], _f32).reshape(2, 1, 1)

    zeros_hbm = jnp.zeros((ZROWS, D), _f32)
    tcomb = _build_tables(h, x, w1ab, b1z, sgn)
    z = _gather_edges(tcomb, row_g2d, col_g2d)

    eye = jnp.eye(DZ, dtype=_f32)
    s1 = eye[:, :HID]
    sq = jnp.zeros((DZ, HID), _f32).at[HID:HID + 3].set(
        jnp.broadcast_to(w1c, (3, HID)))
    s2 = eye[:, HID:HID + 4]
    srad = jnp.zeros((DZ, 1), _f32).at[HID:HID + 3].set(1.0)

    u = _edge_mlp(z, s1, sq, s2, srad, msg_W2, msg_b2.reshape(1, HID),
                  coord_W1, coord_b1.reshape(1, HID), coord_W2,
                  coord_b2.reshape(1, 1))

    p = _scatter_edges(u, row_s2d, zeros_hbm)

    w1f_ext = jnp.zeros((DZ, HID), _f32).at[:HID].set(node_W1[:HID])
    sx = jnp.zeros((DZ, 3), _f32).at[HID:HID + 3].set(jnp.eye(3, dtype=_f32))

    x_new, h_new = _node_mlp(p, h, x, w1f_ext, node_W1[HID:],
                             node_b1.reshape(1, HID), node_W2,
                             node_b2.reshape(1, EMB), sx)
    return (x_new, h_new)


# speedup vs baseline: 1.6753x; 1.1949x over previous
import jax
import jax.numpy as jnp
from jax import lax
from jax.experimental import pallas as pl
from jax.experimental.pallas import tpu as pltpu
from jax.experimental.pallas import tpu_sc as plsc

N = 10000
E = 320000
EMB = 128
HID = 64
D = 80
DZ = 128
CHUNK = 128
CHUNK_G = 256
NC = 2
NS = 16
NW = NC * NS
KPT = 80
NCHUNKS = NW * KPT
EPAD = NCHUNKS * CHUNK
KPT_G = EPAD // (NW * CHUNK_G)
NSLOT = 4
N_ACC = 10240
ROWS_PT = N_ACC // NS
ZROWS = 128

_f32 = jnp.float32


def _tables_body(h_ref, x_ref, w_ref, b_ref, sgn_ref, t_ref):
    h = h_ref[...]
    nb = h.shape[0]
    pad = jnp.zeros((nb, D - HID - 3), _f32)
    a = jnp.dot(h, w_ref[0], preferred_element_type=_f32) + b_ref[0]
    t_ref[...] = jnp.concatenate([a, sgn_ref[0, 0, 0] * x_ref[...], pad],
                                 axis=1)


def _build_tables(h, x, w1ab, b1z, sgn):
    nb = 2000
    grid = N // nb
    return pl.pallas_call(
        _tables_body,
        grid=(2, grid),
        in_specs=[
            pl.BlockSpec((nb, EMB), lambda t, i: (i, 0)),
            pl.BlockSpec((nb, 3), lambda t, i: (i, 0)),
            pl.BlockSpec((1, EMB, HID), lambda t, i: (t, 0, 0)),
            pl.BlockSpec((1, 1, HID), lambda t, i: (t, 0, 0)),
            pl.BlockSpec((1, 1, 1), lambda t, i: (t, 0, 0)),
        ],
        out_specs=pl.BlockSpec((nb, D), lambda t, i: (t * grid + i, 0)),
        out_shape=jax.ShapeDtypeStruct((2 * N, D), _f32),
    )(h, x, w1ab, b1z, sgn)


def _make_gather_body(kpt_g, base_chunk):
    kpt_c0 = kpt_g * 8 // 5
    kpt_c1 = 2 * kpt_g - kpt_c0

    def _gather_body(tcomb, row2d, col2d, z_out, idx_r, idx_c, zbufs,
                     g1, g2, w):
        cid = lax.axis_index("c")
        sid = lax.axis_index("s")
        chunk0 = jnp.where(cid == 0, sid * kpt_c0,
                           NS * kpt_c0 + sid * kpt_c1)
        kpt_t = jnp.where(cid == 0, kpt_c0, kpt_c1)

        pltpu.sync_copy(row2d.at[pl.ds(base_chunk + chunk0, kpt_c0)], idx_r)
        pltpu.sync_copy(col2d.at[pl.ds(base_chunk + chunk0, kpt_c0)], idx_c)

        def g1_start(k, s):
            pltpu.async_copy(tcomb.at[idx_r.at[k]], zbufs.at[s], g1.at[s])

        def g1_wait(k, s):
            pltpu.make_async_copy(tcomb.at[idx_r.at[k]], zbufs.at[s],
                                  g1.at[s]).wait()

        def g2_start(k, s):
            pltpu.async_copy(tcomb.at[idx_c.at[k]], zbufs.at[s], g2.at[s],
                             add=True)

        def g2_wait(k, s):
            pltpu.make_async_copy(tcomb.at[idx_c.at[k]], zbufs.at[s],
                                  g2.at[s]).wait()

        def _wdst(k):
            base = (chunk0 + k) * CHUNK_G
            return z_out.at[pl.ds(base, CHUNK_G)].at[:, pl.ds(0, D)]

        def w_start(k, s):
            pltpu.async_copy(zbufs.at[s], _wdst(k), w.at[s])

        def w_wait(k, s):
            pltpu.make_async_copy(zbufs.at[s], _wdst(k), w.at[s]).wait()

        g1_start(0, 0)

        def body(kb, carry):
            for j in range(NSLOT):
                k = kb * NSLOT + j
                s = j
                snext = (j + 1) % NSLOT
                g1_wait(k, s)
                g2_start(k, s)

                @pl.when(k + 1 < kpt_t)
                def _():
                    @pl.when(k >= NSLOT - 1)
                    def _():
                        w_wait(k - (NSLOT - 1), snext)

                    g1_start(k + 1, snext)

                g2_wait(k, s)
                w_start(k, s)
            return carry

        lax.fori_loop(0, kpt_t // NSLOT, body, 0)
        for s in range(NSLOT):
            w_wait(kpt_t - NSLOT + s, s)

    return _gather_body


def _gather_edges(tcomb, row2d, col2d, base_chunk, nchunk):
    kpt_g = nchunk // NW
    mesh = plsc.VectorSubcoreMesh(
        core_axis_name="c", subcore_axis_name="s", num_cores=NC, num_subcores=NS
    )
    return pl.kernel(
        _make_gather_body(kpt_g, base_chunk),
        out_type=jax.ShapeDtypeStruct((nchunk * CHUNK_G, DZ), _f32),
        mesh=mesh,
        compiler_params=pltpu.CompilerParams(use_tc_tiling_on_sc=False),
        scratch_types=[
            pltpu.VMEM((kpt_g * 8 // 5, CHUNK_G), jnp.int32),
            pltpu.VMEM((kpt_g * 8 // 5, CHUNK_G), jnp.int32),
            pltpu.VMEM((NSLOT, CHUNK_G, D), _f32),
            pltpu.SemaphoreType.DMA((NSLOT,)),
            pltpu.SemaphoreType.DMA((NSLOT,)),
            pltpu.SemaphoreType.DMA((NSLOT,)),
        ],
    )(tcomb, row2d, col2d)


def _edge_mlp_body(z_ref, s1_ref, sq_ref, s2_ref, srad_ref, w2_ref, b2_ref,
                   cw1_ref, cb1_ref, cw2_ref, cb2_ref, u_ref):
    zr = z_ref[...]
    nb = zr.shape[0]
    lane = lax.broadcasted_iota(jnp.int32, (nb, DZ), 1)
    z = jnp.where(lane < D, zr, 0.0)
    zz = z * z
    rad = jnp.dot(zz, srad_ref[...], preferred_element_type=_f32)
    z1 = jax.nn.relu(
        jnp.dot(z, s1_ref[...], preferred_element_type=_f32)
        + jnp.dot(zz, sq_ref[...], preferred_element_type=_f32)
    )
    msgs = jax.nn.relu(
        jnp.dot(z1, w2_ref[...], preferred_element_type=_f32) + b2_ref[...]
    )
    t = jax.nn.relu(
        jnp.dot(msgs, cw1_ref[...], preferred_element_type=_f32) + cb1_ref[...]
    )
    cc = jnp.dot(t, cw2_ref[...], preferred_element_type=_f32) + cb2_ref[...]
    scale = cc * lax.rsqrt(rad + 1e-8)
    cu = jnp.dot(z, s2_ref[...], preferred_element_type=_f32) * scale
    pad = jnp.zeros((nb, DZ - HID - 4), _f32)
    u_ref[...] = jnp.concatenate([msgs, cu, pad], axis=1)


def _edge_mlp(z, s1, sq, s2, srad, w2, b2, cw1, cb1, cw2, cb2):
    eb = 2048
    grid = z.shape[0] // eb
    full = lambda r, c: pl.BlockSpec((r, c), lambda i: (0, 0))
    return pl.pallas_call(
        _edge_mlp_body,
        grid=(grid,),
        in_specs=[
            pl.BlockSpec((eb, DZ), lambda i: (i, 0)),
            full(DZ, HID), full(DZ, HID), full(DZ, 4), full(DZ, 1),
            full(HID, HID), full(1, HID),
            full(HID, HID), full(1, HID), full(HID, 1), full(1, 1),
        ],
        out_specs=pl.BlockSpec((eb, DZ), lambda i: (i, 0)),
        out_shape=jax.ShapeDtypeStruct((z.shape[0], DZ), _f32),
    )(z, s1, sq, s2, srad, w2, b2, cw1, cb1, cw2, cb2)


def _make_scatter_body(kpt):
    def _scatter_body(u, row2d, zeros_hbm, p_out, ubufs, idxb, zbuf, acc,
                      r, ss):
        cid = lax.axis_index("c")
        sid = lax.axis_index("s")
        wid = sid * NC + cid
        chunk0 = wid * kpt
        row0 = sid * ROWS_PT

        pltpu.sync_copy(zeros_hbm, zbuf)
        for j in range(ROWS_PT // ZROWS):
            pltpu.sync_copy(zbuf, acc.at[pl.ds(row0 + j * ZROWS, ZROWS)])
        pltpu.sync_copy(row2d.at[pl.ds(chunk0, kpt)], idxb)
        plsc.subcore_barrier()

        def _rsrc(k):
            base = (chunk0 + k) * CHUNK
            return u.at[pl.ds(base, CHUNK)].at[:, pl.ds(0, D)]

        def r_start(k, s):
            pltpu.async_copy(_rsrc(k), ubufs.at[s], r.at[s])

        def r_wait(k, s):
            pltpu.make_async_copy(_rsrc(k), ubufs.at[s], r.at[s]).wait()

        def s_start(k, s):
            pltpu.async_copy(ubufs.at[s], acc.at[idxb.at[k]], ss.at[s],
                             add=True)

        def s_wait(k, s):
            pltpu.make_async_copy(ubufs.at[s], acc.at[idxb.at[k]],
                                  ss.at[s]).wait()

        r_start(0, 0)

        def body(kb, carry):
            for j in range(NSLOT):
                k = kb * NSLOT + j
                s = j
                snext = (j + 1) % NSLOT
                r_wait(k, s)
                s_start(k, s)

                @pl.when(k + 1 < kpt)
                def _():
                    @pl.when(k >= NSLOT - 1)
                    def _():
                        s_wait(k - (NSLOT - 1), snext)

                    r_start(k + 1, snext)
            return carry

        lax.fori_loop(0, kpt // NSLOT, body, 0)
        for s in range(NSLOT):
            s_wait(kpt - NSLOT + s, s)
        plsc.subcore_barrier()

        for j in range(ROWS_PT // ZROWS):
            rr = row0 + j * ZROWS
            pltpu.sync_copy(acc.at[pl.ds(rr, ZROWS)], zbuf)
            pltpu.sync_copy(
                zbuf, p_out.at[cid].at[pl.ds(rr, ZROWS)].at[:, pl.ds(0, D)])

    return _scatter_body


def _scatter_edges(u, row2d, zeros_hbm):
    kpt = row2d.shape[0] // NW
    mesh = plsc.VectorSubcoreMesh(
        core_axis_name="c", subcore_axis_name="s", num_cores=NC, num_subcores=NS
    )
    return pl.kernel(
        _make_scatter_body(kpt),
        out_type=jax.ShapeDtypeStruct((NC, N_ACC, DZ), _f32),
        mesh=mesh,
        compiler_params=pltpu.CompilerParams(use_tc_tiling_on_sc=False),
        scratch_types=[
            pltpu.VMEM((NSLOT, CHUNK, D), _f32),
            pltpu.VMEM((kpt, CHUNK), jnp.int32),
            pltpu.VMEM((ZROWS, D), _f32),
            pltpu.VMEM_SHARED((N_ACC, D), _f32),
            pltpu.SemaphoreType.DMA((NSLOT,)),
            pltpu.SemaphoreType.DMA((NSLOT,)),
        ],
    )(u, row2d, zeros_hbm)


def _node_mlp_body(p_ref, q_ref, h_ref, x_ref, w1f_ref, w1h_ref, b1_ref,
                   w2_ref, b2_ref, sx_ref, xo_ref, ho_ref):
    gr = p_ref[0] + p_ref[1] + q_ref[0] + q_ref[1]
    lane = lax.broadcasted_iota(jnp.int32, gr.shape, 1)
    g = jnp.where(lane < D, gr, 0.0)
    h = h_ref[...]
    t = jax.nn.relu(
        jnp.dot(g, w1f_ref[...], preferred_element_type=_f32)
        + jnp.dot(h, w1h_ref[...], preferred_element_type=_f32)
        + b1_ref[...]
    )
    ho_ref[...] = h + jnp.dot(t, w2_ref[...], preferred_element_type=_f32) \
        + b2_ref[...]
    xo_ref[...] = x_ref[...] + jnp.dot(g, sx_ref[...],
                                       preferred_element_type=_f32)


def _node_mlp(p, q, h, x, w1f_ext, w1h, b1, w2, b2, sx):
    nb = 2000
    grid = N // nb
    full = lambda r, c: pl.BlockSpec((r, c), lambda i: (0, 0))
    return pl.pallas_call(
        _node_mlp_body,
        grid=(grid,),
        in_specs=[
            pl.BlockSpec((NC, nb, DZ), lambda i: (0, i, 0)),
            pl.BlockSpec((NC, nb, DZ), lambda i: (0, i, 0)),
            pl.BlockSpec((nb, EMB), lambda i: (i, 0)),
            pl.BlockSpec((nb, 3), lambda i: (i, 0)),
            full(DZ, HID), full(EMB, HID), full(1, HID),
            full(HID, EMB), full(1, EMB), full(DZ, 3),
        ],
        out_specs=[
            pl.BlockSpec((nb, 3), lambda i: (i, 0)),
            pl.BlockSpec((nb, EMB), lambda i: (i, 0)),
        ],
        out_shape=[
            jax.ShapeDtypeStruct((N, 3), _f32),
            jax.ShapeDtypeStruct((N, EMB), _f32),
        ],
    )(p, q, h, x, w1f_ext, w1h, b1, w2, b2, sx)


def kernel(x, h, edge_index, msg_W1, msg_b1, msg_W2, msg_b2, coord_W1,
           coord_b1, coord_W2, coord_b2, node_W1, node_b1, node_W2, node_b2):
    ei = edge_index.astype(jnp.int32)
    row = ei[0]
    col = ei[1]
    npad = EPAD - E
    zpad = jnp.zeros((npad,), jnp.int32)
    gpad = jnp.zeros((32, CHUNK_G), jnp.int32)
    row_g2d = jnp.concatenate([
        jnp.concatenate([row, zpad]).reshape(EPAD // CHUNK_G, CHUNK_G), gpad])
    col_g2d = jnp.concatenate([
        (jnp.concatenate([col, zpad]) + N).reshape(EPAD // CHUNK_G, CHUNK_G),
        gpad])
    dummy = N + (jnp.arange(npad, dtype=jnp.int32) % (N_ACC - N))
    row_s2d = jnp.concatenate([row, dummy]).reshape(NCHUNKS, CHUNK)

    w1ab = jnp.stack([msg_W1[:EMB], msg_W1[EMB:2 * EMB]])
    w1c = msg_W1[2 * EMB]
    b1z = jnp.stack([msg_b1.reshape(1, HID), jnp.zeros((1, HID), _f32)])
    sgn = jnp.array([1.0, -1.0], _f32).reshape(2, 1, 1)

    zeros_hbm = jnp.zeros((ZROWS, D), _f32)
    tcomb = _build_tables(h, x, w1ab, b1z, sgn)
    hg = EPAD // CHUNK_G // 2
    hs = NCHUNKS // 2
    z_a = _gather_edges(tcomb, row_g2d, col_g2d, 0, hg)
    z_b = _gather_edges(tcomb, row_g2d, col_g2d, hg, hg)

    eye = jnp.eye(DZ, dtype=_f32)
    s1 = eye[:, :HID]
    sq = jnp.zeros((DZ, HID), _f32).at[HID:HID + 3].set(
        jnp.broadcast_to(w1c, (3, HID)))
    s2 = eye[:, HID:HID + 4]
    srad = jnp.zeros((DZ, 1), _f32).at[HID:HID + 3].set(1.0)

    mlp_args = (s1, sq, s2, srad, msg_W2, msg_b2.reshape(1, HID),
                coord_W1, coord_b1.reshape(1, HID), coord_W2,
                coord_b2.reshape(1, 1))
    u_a = _edge_mlp(z_a, *mlp_args)
    u_b = _edge_mlp(z_b, *mlp_args)

    p_a = _scatter_edges(u_a, row_s2d[:hs], zeros_hbm)
    p_b = _scatter_edges(u_b, row_s2d[hs:], zeros_hbm)

    w1f_ext = jnp.zeros((DZ, HID), _f32).at[:HID].set(node_W1[:HID])
    sx = jnp.zeros((DZ, 3), _f32).at[HID:HID + 3].set(jnp.eye(3, dtype=_f32))

    x_new, h_new = _node_mlp(p_a, p_b, h, x, w1f_ext, node_W1[HID:],
                             node_b1.reshape(1, HID), node_W2,
                             node_b2.reshape(1, EMB), sx)
    return (x_new, h_new)
